# baseline (speedup 1.0000x reference)
import functools

import jax
import jax.numpy as jnp
from jax import lax
from jax.experimental import pallas as pl
from jax.experimental.pallas import tpu as pltpu
from jax.experimental.pallas import tpu_sc as plsc

_N = 10000
_E = 320000
_D = 128
_DE = 16

_NC = 2
_NS = 16
_NW = _NC * _NS
_BLK = 256
_NBLK = 40
_WB = 128
_NPAIR = _NBLK // 2
_EPT = _NBLK * _BLK
_EPAD = _NW * _EPT
_NACC = 10240
_NTAB = _N + 16
_RPT = _NACC // _NS


_DH = _D // 2


def _ab_body(nf_ref, wa_ref, wb_ref, a0_ref, a1_ref, b0_ref, b1_ref):
    x = nf_ref[...]
    a = jnp.dot(x, wa_ref[...], preferred_element_type=jnp.float32)
    b = jnp.dot(x, wb_ref[...], preferred_element_type=jnp.float32)
    a0_ref[...] = a[:, :_DH]
    a1_ref[...] = a[:, _DH:]
    b0_ref[...] = b[:, :_DH]
    b1_ref[...] = b[:, _DH:]


_ab_call = pl.pallas_call(
    _ab_body,
    grid=(5,),
    in_specs=[
        pl.BlockSpec((2000, _D), lambda i: (i, 0)),
        pl.BlockSpec((_D, _D), lambda i: (0, 0)),
        pl.BlockSpec((_D, _D), lambda i: (0, 0)),
    ],
    out_specs=[pl.BlockSpec((2000, _DH), lambda i: (i, 0))] * 4,
    out_shape=[jax.ShapeDtypeStruct((_N, _DH), jnp.float32)] * 4,
)


def _c_body(ef_ref, wc_ref, b1_ref, c_ref):
    c = (jnp.dot(ef_ref[...], wc_ref[...], preferred_element_type=jnp.float32)
         + b1_ref[...])
    c_ref[0] = c[:, :_DH]
    c_ref[1] = c[:, _DH:]


_c_call = pl.pallas_call(
    _c_body,
    grid=(_EPAD // 4096,),
    in_specs=[
        pl.BlockSpec((4096, _DE), lambda i: (i, 0)),
        pl.BlockSpec((_DE, _D), lambda i: (0, 0)),
        pl.BlockSpec((1, _D), lambda i: (0, 0)),
    ],
    out_specs=pl.BlockSpec((2, 4096, _DH), lambda i: (0, i, 0)),
    out_shape=jax.ShapeDtypeStruct((2, _EPAD, _DH), jnp.float32),
)


def _attn_body(q_ref, k_ref, o_ref):
    q = q_ref[0]
    k = k_ref[0]
    qb = 1000
    m = jnp.full((qb, 1), -1e30, jnp.float32)
    l = jnp.zeros((qb, 1), jnp.float32)
    acc = jnp.zeros((qb, _D), jnp.float32)
    for j in range(5):
        kj = k[j * qb:(j + 1) * qb]
        s = lax.dot_general(q, kj, (((1,), (1,)), ((), ())),
                            preferred_element_type=jnp.float32)
        mj = jnp.max(s, axis=1, keepdims=True)
        mn = jnp.maximum(m, mj)
        corr = jnp.exp(m - mn)
        p = jnp.exp(s - mn)
        l = l * corr + jnp.sum(p, axis=1, keepdims=True)
        acc = acc * corr + jnp.dot(p, kj, preferred_element_type=jnp.float32)
        m = mn
    o_ref[0] = q - acc / l


_attn_call = pl.pallas_call(
    _attn_body,
    grid=(10,),
    in_specs=[
        pl.BlockSpec((1, 1000, _D), lambda i: (i // 5, i % 5, 0)),
        pl.BlockSpec((1, _N // 2, _D), lambda i: (1 - i // 5, 0, 0)),
    ],
    out_specs=pl.BlockSpec((1, 1000, _D), lambda i: (i // 5, i % 5, 0)),
    out_shape=jax.ShapeDtypeStruct((2, _N // 2, _D), jnp.float32),
)


@functools.cache
def _get_sc_scatter():
  mesh = plsc.VectorSubcoreMesh(
      core_axis_name="c", subcore_axis_name="s",
      num_cores=_NC, num_subcores=_NS)

  @functools.partial(
      pl.kernel,
      out_type=(
          jax.ShapeDtypeStruct((2, _NC, _NACC, _DH), jnp.float32),
          jax.ShapeDtypeStruct((_NC, _NACC, 16), jnp.float32),
      ),
      mesh=mesh,
      scratch_types=[
          pltpu.VMEM((_NBLK, _BLK), jnp.int32),
          pltpu.VMEM((_NBLK, _BLK), jnp.int32),
          pltpu.VMEM((_BLK, _DH), jnp.float32),
          pltpu.VMEM((_BLK, _DH), jnp.float32),
          pltpu.VMEM((_BLK, _DH), jnp.float32),
          pltpu.VMEM((_BLK, 16), jnp.float32),
          pltpu.VMEM((_BLK, 16), jnp.float32),
          pltpu.VMEM_SHARED((_NACC, _DH), jnp.float32),
          pltpu.VMEM_SHARED((_NACC, 16), jnp.float32),
          pltpu.SemaphoreType.DMA,
      ],
      compiler_params=pltpu.CompilerParams(use_tc_tiling_on_sc=False),
  )
  def _sc_scatter(a0_hbm, a1_hbm, b0_hbm, b1_hbm, c_hbm, src_hbm, dst_hbm,
                  sh_out, sd_out,
                  src2d, dst2d, a0_v, b0_v, c0_v,
                  sd_v, ones_v, sh_acc, sd_acc, sem0):
    cid = lax.axis_index("c")
    sid = lax.axis_index("s")
    wid = cid * _NS + sid
    row0 = sid * _RPT

    zero16 = jnp.zeros((16,), jnp.float32)
    io = lax.iota(jnp.int32, 16)
    e0 = jnp.where(io == 0, 1.0, 0.0)

    def _init_row(i, carry):
      sd_v[i, :] = zero16
      ones_v[i, :] = e0
      return carry

    lax.fori_loop(0, _BLK, _init_row, 0)

    pltpu.sync_copy(src_hbm.at[wid], src2d)
    pltpu.sync_copy(dst_hbm.at[wid], dst2d)

    def _zero_a(i, carry):
      for j in range(_DH // 16):
        a0_v[i, pl.ds(j * 16, 16)] = zero16
      return carry

    for p, (ah, bh) in enumerate(((a0_hbm, b0_hbm), (a1_hbm, b1_hbm))):

      def _start(blk, slot):
        av, bv, cv, sem = slot
        pltpu.async_copy(ah.at[src2d.at[blk]], av, sem)
        pltpu.async_copy(bh.at[dst2d.at[blk]], bv, sem)
        pltpu.async_copy(c_hbm.at[p, pl.ds(wid * _NBLK * _BLK + blk * _BLK,
                                           _BLK)], cv, sem)

      def _finish(blk, slot, pp):
        av, bv, cv, sem = slot
        pltpu.make_async_copy(ah.at[src2d.at[blk]], av, sem).wait()
        pltpu.make_async_copy(bh.at[dst2d.at[blk]], bv, sem).wait()
        pltpu.make_async_copy(
            c_hbm.at[pp, pl.ds(wid * _NBLK * _BLK + blk * _BLK, _BLK)],
            cv, sem).wait()

        def _row(i, c2):
          for j in range(_DH // 16):
            sl = pl.ds(j * 16, 16)
            av[i, sl] = jnp.maximum(av[i, sl] + bv[i, sl] + cv[i, sl], 0.0)
          return c2

        lax.fori_loop(0, _BLK, _row, 0)
        pltpu.sync_copy(av, sh_acc.at[dst2d.at[blk]], add=True)
        if pp == 0:
          pltpu.sync_copy(ones_v, sd_acc.at[dst2d.at[blk]], add=True)

      lax.fori_loop(0, _BLK, _zero_a, 0)
      for kk in range(_RPT // _WB):
        pltpu.sync_copy(a0_v.at[pl.ds(0, _WB)],
                        sh_acc.at[pl.ds(row0 + kk * _WB, _WB)])
        if p == 0:
          pltpu.sync_copy(sd_v.at[pl.ds(0, _WB)],
                          sd_acc.at[pl.ds(row0 + kk * _WB, _WB)])
      plsc.subcore_barrier()

      def _blk_loop(blk, carry):
        _start(blk, (a0_v, b0_v, c0_v, sem0))
        _finish(blk, (a0_v, b0_v, c0_v, sem0), p)
        return carry

      lax.fori_loop(0, _NBLK, _blk_loop, 0)
      plsc.subcore_barrier()

      for kk in range(_RPT // _WB):
        r = row0 + kk * _WB
        pltpu.sync_copy(sh_acc.at[pl.ds(r, _WB)], a0_v.at[pl.ds(0, _WB)])
        pltpu.sync_copy(a0_v.at[pl.ds(0, _WB)],
                        sh_out.at[p, cid, pl.ds(r, _WB)])
        if p == 0:
          pltpu.sync_copy(sd_acc.at[pl.ds(r, _WB)], sd_v.at[pl.ds(0, _WB)])
          pltpu.sync_copy(sd_v.at[pl.ds(0, _WB)],
                          sd_out.at[cid, pl.ds(r, _WB)])

  return _sc_scatter


def _final_body(sl0, sl1, sh0, sh1, sd0, sd1, u_ref, nf_ref, w2t, b2r,
                wim, wiu, bi, wht, bh, o_ref):
    s_lo = sl0[...] + sl1[...]
    s_hi = sh0[...] + sh1[...]
    deg = (sd0[...] + sd1[...])[:, 0:1]
    m = (jnp.dot(s_lo, w2t[:_DH], preferred_element_type=jnp.float32)
         + jnp.dot(s_hi, w2t[_DH:], preferred_element_type=jnp.float32)
         + deg * b2r[...])
    u = u_ref[...]
    nf = nf_ref[...]
    gi = (jnp.dot(m, wim[...], preferred_element_type=jnp.float32)
          + jnp.dot(u, wiu[...], preferred_element_type=jnp.float32)
          + bi[...])
    gh = jnp.dot(nf, wht[...], preferred_element_type=jnp.float32) + bh[...]
    r = jax.nn.sigmoid(gi[:, :_D] + gh[:, :_D])
    z = jax.nn.sigmoid(gi[:, _D:2 * _D] + gh[:, _D:2 * _D])
    n = jnp.tanh(gi[:, 2 * _D:] + r * gh[:, 2 * _D:])
    o_ref[...] = (1.0 - z) * n + z * nf


_final_call = pl.pallas_call(
    _final_body,
    grid=(5,),
    in_specs=[
        pl.BlockSpec((2000, _DH), lambda i: (i, 0)),
        pl.BlockSpec((2000, _DH), lambda i: (i, 0)),
        pl.BlockSpec((2000, _DH), lambda i: (i, 0)),
        pl.BlockSpec((2000, _DH), lambda i: (i, 0)),
        pl.BlockSpec((2000, 16), lambda i: (i, 0)),
        pl.BlockSpec((2000, 16), lambda i: (i, 0)),
        pl.BlockSpec((2000, _D), lambda i: (i, 0)),
        pl.BlockSpec((2000, _D), lambda i: (i, 0)),
        pl.BlockSpec((_D, _D), lambda i: (0, 0)),
        pl.BlockSpec((1, _D), lambda i: (0, 0)),
        pl.BlockSpec((_D, 3 * _D), lambda i: (0, 0)),
        pl.BlockSpec((_D, 3 * _D), lambda i: (0, 0)),
        pl.BlockSpec((1, 3 * _D), lambda i: (0, 0)),
        pl.BlockSpec((_D, 3 * _D), lambda i: (0, 0)),
        pl.BlockSpec((1, 3 * _D), lambda i: (0, 0)),
    ],
    out_specs=pl.BlockSpec((2000, _D), lambda i: (i, 0)),
    out_shape=jax.ShapeDtypeStruct((_N, _D), jnp.float32),
)


def kernel(node_features, edge_features, edge_index, W1, b1, W2, b2,
           W_ih, b_ih, W_hh, b_hh):
    nf = node_features
    src = edge_index[0]
    dst = edge_index[1]
    pad_e = _EPAD - _E
    srcp = jnp.concatenate([src, jnp.full((pad_e,), _N, jnp.int32)])
    dstp = jnp.concatenate([dst, jnp.full((pad_e,), _N, jnp.int32)])
    efp = jnp.concatenate(
        [edge_features, jnp.zeros((pad_e, _DE), jnp.float32)])

    w1at = W1[:, :_D].T
    w1bt = W1[:, _D:2 * _D].T
    w1ct = W1[:, 2 * _D:].T

    a0, a1, b0, b1t = _ab_call(nf, w1at, w1bt)
    zpad = jnp.zeros((_NTAB - _N, _DH), jnp.float32)
    a0 = jnp.concatenate([a0, zpad])
    a1 = jnp.concatenate([a1, zpad])
    b0 = jnp.concatenate([b0, zpad])
    b1t = jnp.concatenate([b1t, zpad])
    c_rows = _c_call(efp, w1ct, b1.reshape(1, _D))

    sh, sd = _get_sc_scatter()(
        a0, a1, b0, b1t, c_rows,
        srcp.reshape(_NW, _NBLK, _BLK), dstp.reshape(_NW, _NBLK, _BLK))

    xy = nf.reshape(2, _N // 2, _D)
    u = _attn_call(xy, xy).reshape(_N, _D)

    wiht = W_ih.T
    h_new = _final_call(
        sh[0, 0, :_N], sh[0, 1, :_N], sh[1, 0, :_N], sh[1, 1, :_N],
        sd[0, :_N], sd[1, :_N], u, nf,
        W2.T, b2.reshape(1, _D), wiht[:_D], wiht[_D:],
        b_ih.reshape(1, 3 * _D), W_hh.T, b_hh.reshape(1, 3 * _D))
    return h_new


# speedup vs baseline: 1.2475x; 1.2475x over previous
import functools

import jax
import jax.numpy as jnp
from jax import lax
from jax.experimental import pallas as pl
from jax.experimental.pallas import tpu as pltpu
from jax.experimental.pallas import tpu_sc as plsc

_N = 10000
_E = 320000
_D = 128
_DE = 16

_NC = 2
_NS = 16
_NW = _NC * _NS
_BLK = 128
_NBLK = 80
_NSLOT = 2
_NGRP = _NBLK // _NSLOT
_WB = 128
_NPAIR = _NBLK // 2
_EPT = _NBLK * _BLK
_EPAD = _NW * _EPT
_NACC = 10240
_NTAB = _N + 16
_RPT = _NACC // _NS


_DH = _D // 2


def _ab_body(nf_ref, wa_ref, wb_ref, a0_ref, a1_ref, b0_ref, b1_ref):
    x = nf_ref[...]
    a = jnp.dot(x, wa_ref[...], preferred_element_type=jnp.float32)
    b = jnp.dot(x, wb_ref[...], preferred_element_type=jnp.float32)
    a0_ref[...] = a[:, :_DH]
    a1_ref[...] = a[:, _DH:]
    b0_ref[...] = b[:, :_DH]
    b1_ref[...] = b[:, _DH:]


_ab_call = pl.pallas_call(
    _ab_body,
    grid=(5,),
    in_specs=[
        pl.BlockSpec((2000, _D), lambda i: (i, 0)),
        pl.BlockSpec((_D, _D), lambda i: (0, 0)),
        pl.BlockSpec((_D, _D), lambda i: (0, 0)),
    ],
    out_specs=[pl.BlockSpec((2000, _DH), lambda i: (i, 0))] * 4,
    out_shape=[jax.ShapeDtypeStruct((_N, _DH), jnp.float32)] * 4,
)


def _c_body(ef_ref, wc_ref, b1_ref, c_ref):
    c = (jnp.dot(ef_ref[...], wc_ref[...], preferred_element_type=jnp.float32)
         + b1_ref[...])
    c_ref[0] = c[:, :_DH]
    c_ref[1] = c[:, _DH:]


_c_call = pl.pallas_call(
    _c_body,
    grid=(_EPAD // 4096,),
    in_specs=[
        pl.BlockSpec((4096, _DE), lambda i: (i, 0)),
        pl.BlockSpec((_DE, _D), lambda i: (0, 0)),
        pl.BlockSpec((1, _D), lambda i: (0, 0)),
    ],
    out_specs=pl.BlockSpec((2, 4096, _DH), lambda i: (0, i, 0)),
    out_shape=jax.ShapeDtypeStruct((2, _EPAD, _DH), jnp.float32),
)


def _attn_body(q_ref, k_ref, o_ref):
    q = q_ref[0]
    k = k_ref[0]
    qb = 1000
    m = jnp.full((qb, 1), -1e30, jnp.float32)
    l = jnp.zeros((qb, 1), jnp.float32)
    acc = jnp.zeros((qb, _D), jnp.float32)
    for j in range(5):
        kj = k[j * qb:(j + 1) * qb]
        s = lax.dot_general(q, kj, (((1,), (1,)), ((), ())),
                            preferred_element_type=jnp.float32)
        mj = jnp.max(s, axis=1, keepdims=True)
        mn = jnp.maximum(m, mj)
        corr = jnp.exp(m - mn)
        p = jnp.exp(s - mn)
        l = l * corr + jnp.sum(p, axis=1, keepdims=True)
        acc = acc * corr + jnp.dot(p, kj, preferred_element_type=jnp.float32)
        m = mn
    o_ref[0] = q - acc / l


_attn_call = pl.pallas_call(
    _attn_body,
    grid=(10,),
    in_specs=[
        pl.BlockSpec((1, 1000, _D), lambda i: (i // 5, i % 5, 0)),
        pl.BlockSpec((1, _N // 2, _D), lambda i: (1 - i // 5, 0, 0)),
    ],
    out_specs=pl.BlockSpec((1, 1000, _D), lambda i: (i // 5, i % 5, 0)),
    out_shape=jax.ShapeDtypeStruct((2, _N // 2, _D), jnp.float32),
)


@functools.cache
def _get_sc_scatter():
  mesh = plsc.VectorSubcoreMesh(
      core_axis_name="c", subcore_axis_name="s",
      num_cores=_NC, num_subcores=_NS)

  @functools.partial(
      pl.kernel,
      out_type=(
          jax.ShapeDtypeStruct((2, _NC, _NACC, _DH), jnp.float32),
          jax.ShapeDtypeStruct((_NW, _NACC), jnp.float32),
      ),
      mesh=mesh,
      scratch_types=[
          pltpu.VMEM((_NBLK, _BLK), jnp.int32),
          pltpu.VMEM((_NBLK, _BLK), jnp.int32),
      ] + [pltpu.VMEM((_BLK, _DH), jnp.float32)] * (3 * _NSLOT) + [
          pltpu.VMEM((_NACC,), jnp.float32),
          pltpu.VMEM_SHARED((_NACC, _DH), jnp.float32),
      ] + [pltpu.SemaphoreType.DMA] * _NSLOT,
      compiler_params=pltpu.CompilerParams(use_tc_tiling_on_sc=False,
                                          needs_layout_passes=False),
  )
  def _sc_scatter(a0_hbm, a1_hbm, b0_hbm, b1_hbm, c_hbm, src_hbm, dst_hbm,
                  sh_out, deg_out, src2d, dst2d, *rest):
    bufs = rest[:3 * _NSLOT]
    deg_v, sh_acc = rest[3 * _NSLOT:3 * _NSLOT + 2]
    sems = rest[3 * _NSLOT + 2:]
    slots = tuple((bufs[3 * s], bufs[3 * s + 1], bufs[3 * s + 2], sems[s])
                  for s in range(_NSLOT))
    a0_v = bufs[0]
    cid = lax.axis_index("c")
    sid = lax.axis_index("s")
    wid = cid * _NS + sid
    row0 = sid * _RPT

    zero16 = jnp.zeros((16,), jnp.float32)
    one16 = jnp.full((16,), 1.0, jnp.float32)

    def _init_deg(i, carry):
      deg_v[pl.ds(i * 16, 16)] = zero16
      return carry

    lax.fori_loop(0, _NACC // 16, _init_deg, 0)

    pltpu.sync_copy(src_hbm.at[wid], src2d)
    pltpu.sync_copy(dst_hbm.at[wid], dst2d)

    def _zero_a(i, carry):
      for j in range(_DH // 16):
        a0_v[i, pl.ds(j * 16, 16)] = zero16
      return carry

    for p, (ah, bh) in enumerate(((a0_hbm, b0_hbm), (a1_hbm, b1_hbm))):

      def _start(blk, slot):
        av, bv, cv, sem = slot
        pltpu.async_copy(ah.at[src2d.at[blk]], av, sem)
        pltpu.async_copy(bh.at[dst2d.at[blk]], bv, sem)
        pltpu.async_copy(c_hbm.at[p, pl.ds(wid * _NBLK * _BLK + blk * _BLK,
                                           _BLK)], cv, sem)

      def _finish(blk, slot, pp):
        av, bv, cv, sem = slot
        pltpu.make_async_copy(ah.at[src2d.at[blk]], av, sem).wait()
        pltpu.make_async_copy(bh.at[dst2d.at[blk]], bv, sem).wait()
        pltpu.make_async_copy(
            c_hbm.at[pp, pl.ds(wid * _NBLK * _BLK + blk * _BLK, _BLK)],
            cv, sem).wait()

        def _row(i, c2):
          for j in range(_DH // 16):
            sl = pl.ds(j * 16, 16)
            av[i, sl] = jnp.maximum(av[i, sl] + bv[i, sl] + cv[i, sl], 0.0)
          return c2

        lax.fori_loop(0, _BLK, _row, 0)
        pltpu.sync_copy(av, sh_acc.at[dst2d.at[blk]], add=True)
        if pp == 0:
          def _deg(i, c3):
            idx = dst2d[blk, pl.ds(i * 16, 16)]
            plsc.addupdate_scatter(deg_v, [idx], one16)
            return c3

          lax.fori_loop(0, _BLK // 16, _deg, 0)

      lax.fori_loop(0, _BLK, _zero_a, 0)
      for kk in range(_RPT // _WB):
        pltpu.sync_copy(a0_v.at[pl.ds(0, _WB)],
                        sh_acc.at[pl.ds(row0 + kk * _WB, _WB)])
      plsc.subcore_barrier()

      for s in range(_NSLOT - 1):
        _start(s, slots[s])

      def _grp(g, carry):
        for s in range(_NSLOT):
          blk = g * _NSLOT + s
          nxt = blk + _NSLOT - 1

          @pl.when(nxt < _NBLK)
          def _():
            _start(nxt, slots[(s + _NSLOT - 1) % _NSLOT])

          _finish(blk, slots[s], p)
        return carry

      lax.fori_loop(0, _NGRP, _grp, 0)
      plsc.subcore_barrier()

      for kk in range(_RPT // _WB):
        r = row0 + kk * _WB
        pltpu.sync_copy(sh_acc.at[pl.ds(r, _WB)], a0_v.at[pl.ds(0, _WB)])
        pltpu.sync_copy(a0_v.at[pl.ds(0, _WB)],
                        sh_out.at[p, cid, pl.ds(r, _WB)])
      if p == 0:
        pltpu.sync_copy(deg_v, deg_out.at[wid])

  return _sc_scatter


def _final_body(sl0, sl1, sh0, sh1, dall, u_ref, nf_ref, w2t, b2r,
                wim, wiu, bi, wht, bh, o_ref):
    s_lo = sl0[...] + sl1[...]
    s_hi = sh0[...] + sh1[...]
    deg = jnp.sum(dall[...], axis=1)[:, None]
    m = (jnp.dot(s_lo, w2t[:_DH], preferred_element_type=jnp.float32)
         + jnp.dot(s_hi, w2t[_DH:], preferred_element_type=jnp.float32)
         + deg * b2r[...])
    u = u_ref[...]
    nf = nf_ref[...]
    gi = (jnp.dot(m, wim[...], preferred_element_type=jnp.float32)
          + jnp.dot(u, wiu[...], preferred_element_type=jnp.float32)
          + bi[...])
    gh = jnp.dot(nf, wht[...], preferred_element_type=jnp.float32) + bh[...]
    r = jax.nn.sigmoid(gi[:, :_D] + gh[:, :_D])
    z = jax.nn.sigmoid(gi[:, _D:2 * _D] + gh[:, _D:2 * _D])
    n = jnp.tanh(gi[:, 2 * _D:] + r * gh[:, 2 * _D:])
    o_ref[...] = (1.0 - z) * n + z * nf


_final_call = pl.pallas_call(
    _final_body,
    grid=(5,),
    in_specs=[
        pl.BlockSpec((2000, _DH), lambda i: (i, 0)),
        pl.BlockSpec((2000, _DH), lambda i: (i, 0)),
        pl.BlockSpec((2000, _DH), lambda i: (i, 0)),
        pl.BlockSpec((2000, _DH), lambda i: (i, 0)),
        pl.BlockSpec((2000, _NW), lambda i: (i, 0)),
        pl.BlockSpec((2000, _D), lambda i: (i, 0)),
        pl.BlockSpec((2000, _D), lambda i: (i, 0)),
        pl.BlockSpec((_D, _D), lambda i: (0, 0)),
        pl.BlockSpec((1, _D), lambda i: (0, 0)),
        pl.BlockSpec((_D, 3 * _D), lambda i: (0, 0)),
        pl.BlockSpec((_D, 3 * _D), lambda i: (0, 0)),
        pl.BlockSpec((1, 3 * _D), lambda i: (0, 0)),
        pl.BlockSpec((_D, 3 * _D), lambda i: (0, 0)),
        pl.BlockSpec((1, 3 * _D), lambda i: (0, 0)),
    ],
    out_specs=pl.BlockSpec((2000, _D), lambda i: (i, 0)),
    out_shape=jax.ShapeDtypeStruct((_N, _D), jnp.float32),
)


def kernel(node_features, edge_features, edge_index, W1, b1, W2, b2,
           W_ih, b_ih, W_hh, b_hh):
    nf = node_features
    src = edge_index[0]
    dst = edge_index[1]
    pad_e = _EPAD - _E
    srcp = jnp.concatenate([src, jnp.full((pad_e,), _N, jnp.int32)])
    dstp = jnp.concatenate([dst, jnp.full((pad_e,), _N, jnp.int32)])
    efp = jnp.concatenate(
        [edge_features, jnp.zeros((pad_e, _DE), jnp.float32)])

    w1at = W1[:, :_D].T
    w1bt = W1[:, _D:2 * _D].T
    w1ct = W1[:, 2 * _D:].T

    a0, a1, b0, b1t = _ab_call(nf, w1at, w1bt)
    zpad = jnp.zeros((_NTAB - _N, _DH), jnp.float32)
    a0 = jnp.concatenate([a0, zpad])
    a1 = jnp.concatenate([a1, zpad])
    b0 = jnp.concatenate([b0, zpad])
    b1t = jnp.concatenate([b1t, zpad])
    c_rows = _c_call(efp, w1ct, b1.reshape(1, _D))

    sh, deg_all = _get_sc_scatter()(
        a0, a1, b0, b1t, c_rows,
        srcp.reshape(_NW, _NBLK, _BLK), dstp.reshape(_NW, _NBLK, _BLK))

    xy = nf.reshape(2, _N // 2, _D)
    u = _attn_call(xy, xy).reshape(_N, _D)

    wiht = W_ih.T
    h_new = _final_call(
        sh[0, 0, :_N], sh[0, 1, :_N], sh[1, 0, :_N], sh[1, 1, :_N],
        deg_all[:, :_N].T, u, nf,
        W2.T, b2.reshape(1, _D), wiht[:_D], wiht[_D:],
        b_ih.reshape(1, 3 * _D), W_hh.T, b_hh.reshape(1, 3 * _D))
    return h_new


# speedup vs baseline: 1.7056x; 1.3672x over previous
import functools

import jax
import jax.numpy as jnp
from jax import lax
from jax.experimental import pallas as pl
from jax.experimental.pallas import tpu as pltpu
from jax.experimental.pallas import tpu_sc as plsc

_N = 10000
_E = 320000
_D = 128
_DE = 16

_NC = 2
_NS = 16
_NW = _NC * _NS
_BLK = 128
_NBLK = 80
_NSLOT = 2
_NGRP = _NBLK // _NSLOT
_EPT = _NBLK * _BLK
_EPAD = _NW * _EPT
_NACC = 10240
_NTAB = _N + 16
_RPT = _NACC // _NS
_WB = 128


def _ab_body(nf_ref, wa_ref, wb_ref, a_ref, b_ref):
    x = nf_ref[...]
    a = jnp.dot(x, wa_ref[...], preferred_element_type=jnp.float32)
    b = jnp.dot(x, wb_ref[...], preferred_element_type=jnp.float32)
    a_ref[...] = a.astype(jnp.bfloat16)
    b_ref[...] = b.astype(jnp.bfloat16)


_ab_call = pl.pallas_call(
    _ab_body,
    grid=(5,),
    in_specs=[
        pl.BlockSpec((2000, _D), lambda i: (i, 0)),
        pl.BlockSpec((_D, _D), lambda i: (0, 0)),
        pl.BlockSpec((_D, _D), lambda i: (0, 0)),
    ],
    out_specs=[pl.BlockSpec((2000, _D), lambda i: (i, 0))] * 2,
    out_shape=[jax.ShapeDtypeStruct((_N, _D), jnp.bfloat16)] * 2,
)


def _c_body(ef_ref, wc_ref, b1_ref, c_ref):
    c = (jnp.dot(ef_ref[...], wc_ref[...], preferred_element_type=jnp.float32)
         + b1_ref[...])
    c_ref[...] = c.astype(jnp.bfloat16)


_c_call = pl.pallas_call(
    _c_body,
    grid=(_EPAD // 4096,),
    in_specs=[
        pl.BlockSpec((4096, _DE), lambda i: (i, 0)),
        pl.BlockSpec((_DE, _D), lambda i: (0, 0)),
        pl.BlockSpec((1, _D), lambda i: (0, 0)),
    ],
    out_specs=pl.BlockSpec((4096, _D), lambda i: (i, 0)),
    out_shape=jax.ShapeDtypeStruct((_EPAD, _D), jnp.bfloat16),
)


def _attn_body(q_ref, k_ref, o_ref):
    q = q_ref[0]
    k = k_ref[0]
    qb = 1000
    m = jnp.full((qb, 1), -1e30, jnp.float32)
    l = jnp.zeros((qb, 1), jnp.float32)
    acc = jnp.zeros((qb, _D), jnp.float32)
    for j in range(5):
        kj = k[j * qb:(j + 1) * qb]
        s = lax.dot_general(q, kj, (((1,), (1,)), ((), ())),
                            preferred_element_type=jnp.float32)
        mj = jnp.max(s, axis=1, keepdims=True)
        mn = jnp.maximum(m, mj)
        corr = jnp.exp(m - mn)
        p = jnp.exp(s - mn)
        l = l * corr + jnp.sum(p, axis=1, keepdims=True)
        acc = acc * corr + jnp.dot(p, kj, preferred_element_type=jnp.float32)
        m = mn
    o_ref[0] = q - acc / l


_attn_call = pl.pallas_call(
    _attn_body,
    grid=(10,),
    in_specs=[
        pl.BlockSpec((1, 1000, _D), lambda i: (i // 5, i % 5, 0)),
        pl.BlockSpec((1, _N // 2, _D), lambda i: (1 - i // 5, 0, 0)),
    ],
    out_specs=pl.BlockSpec((1, 1000, _D), lambda i: (i // 5, i % 5, 0)),
    out_shape=jax.ShapeDtypeStruct((2, _N // 2, _D), jnp.float32),
)


@functools.cache
def _get_sc_scatter():
  mesh = plsc.VectorSubcoreMesh(
      core_axis_name="c", subcore_axis_name="s",
      num_cores=_NC, num_subcores=_NS)

  @functools.partial(
      pl.kernel,
      out_type=(
          jax.ShapeDtypeStruct((_NC, _NACC, _D), jnp.bfloat16),
          jax.ShapeDtypeStruct((_NW, _NACC), jnp.float32),
      ),
      mesh=mesh,
      scratch_types=[
          pltpu.VMEM((_NBLK, _BLK), jnp.int32),
          pltpu.VMEM((_NBLK, _BLK), jnp.int32),
      ] + [pltpu.VMEM((_BLK, _D), jnp.bfloat16)] * (3 * _NSLOT) + [
          pltpu.VMEM((_NACC,), jnp.float32),
          pltpu.VMEM_SHARED((_NACC, _D), jnp.bfloat16),
      ] + [pltpu.SemaphoreType.DMA] * _NSLOT,
      compiler_params=pltpu.CompilerParams(use_tc_tiling_on_sc=False,
                                           needs_layout_passes=False),
  )
  def _sc_scatter(a_hbm, b_hbm, c_hbm, src_hbm, dst_hbm,
                  sh_out, deg_out, src2d, dst2d, *rest):
    bufs = rest[:3 * _NSLOT]
    deg_v, sh_acc = rest[3 * _NSLOT:3 * _NSLOT + 2]
    sems = rest[3 * _NSLOT + 2:]
    slots = tuple((bufs[3 * s], bufs[3 * s + 1], bufs[3 * s + 2], sems[s])
                  for s in range(_NSLOT))
    a0_v = bufs[0]

    cid = lax.axis_index("c")
    sid = lax.axis_index("s")
    wid = cid * _NS + sid
    row0 = sid * _RPT

    zero16 = jnp.zeros((16,), jnp.float32)
    zero32b = jnp.zeros((32,), jnp.bfloat16)
    one16 = jnp.full((16,), 1.0, jnp.float32)

    def _init_deg(i, carry):
      deg_v[pl.ds(i * 16, 16)] = zero16
      return carry

    lax.fori_loop(0, _NACC // 16, _init_deg, 0)

    pltpu.sync_copy(src_hbm.at[wid], src2d)
    pltpu.sync_copy(dst_hbm.at[wid], dst2d)

    def _start(blk, slot):
      av, bv, cv, sem = slot
      pltpu.async_copy(a_hbm.at[src2d.at[blk]], av, sem)
      pltpu.async_copy(b_hbm.at[dst2d.at[blk]], bv, sem)
      pltpu.async_copy(c_hbm.at[pl.ds(wid * _NBLK * _BLK + blk * _BLK,
                                      _BLK)], cv, sem)

    def _finish(blk, slot):
      av, bv, cv, sem = slot
      pltpu.make_async_copy(a_hbm.at[src2d.at[blk]], av, sem).wait()
      pltpu.make_async_copy(b_hbm.at[dst2d.at[blk]], bv, sem).wait()
      pltpu.make_async_copy(
          c_hbm.at[pl.ds(wid * _NBLK * _BLK + blk * _BLK, _BLK)],
          cv, sem).wait()

      def _row(i, c2):
        for j in range(_D // 32):
          sl = pl.ds(j * 32, 32)
          av[i, sl] = jnp.maximum(av[i, sl] + bv[i, sl] + cv[i, sl],
                                  jnp.bfloat16(0.0))
        return c2

      lax.fori_loop(0, _BLK, _row, 0)
      pltpu.sync_copy(av, sh_acc.at[dst2d.at[blk]], add=True)

      def _deg(i, c3):
        idx = dst2d[blk, pl.ds(i * 16, 16)]
        plsc.addupdate_scatter(deg_v, [idx], one16)
        return c3

      lax.fori_loop(0, _BLK // 16, _deg, 0)

    def _zero_a(i, carry):
      for j in range(_D // 32):
        a0_v[i, pl.ds(j * 32, 32)] = zero32b
      return carry

    lax.fori_loop(0, _BLK, _zero_a, 0)
    for kk in range(_RPT // _WB):
      pltpu.sync_copy(a0_v.at[pl.ds(0, _WB)],
                      sh_acc.at[pl.ds(row0 + kk * _WB, _WB)])
    plsc.subcore_barrier()

    for s in range(_NSLOT - 1):
      _start(s, slots[s])

    def _grp(g, carry):
      for s in range(_NSLOT):
        blk = g * _NSLOT + s
        nxt = blk + _NSLOT - 1

        @pl.when(nxt < _NBLK)
        def _():
          _start(nxt, slots[(s + _NSLOT - 1) % _NSLOT])

        _finish(blk, slots[s])
      return carry

    lax.fori_loop(0, _NGRP, _grp, 0)
    plsc.subcore_barrier()

    for kk in range(_RPT // _WB):
      r = row0 + kk * _WB
      pltpu.sync_copy(sh_acc.at[pl.ds(r, _WB)], a0_v.at[pl.ds(0, _WB)])
      pltpu.sync_copy(a0_v.at[pl.ds(0, _WB)], sh_out.at[cid, pl.ds(r, _WB)])
    pltpu.sync_copy(deg_v, deg_out.at[wid])

  return _sc_scatter


def _final_body(sh0, sh1, dall, u_ref, nf_ref, w2t, b2r,
                wim, wiu, bi, wht, bh, o_ref):
    s = sh0[...].astype(jnp.float32) + sh1[...].astype(jnp.float32)
    deg = jnp.sum(dall[...], axis=1)[:, None]
    m = (jnp.dot(s, w2t[...], preferred_element_type=jnp.float32)
         + deg * b2r[...])
    u = u_ref[...]
    nf = nf_ref[...]
    gi = (jnp.dot(m, wim[...], preferred_element_type=jnp.float32)
          + jnp.dot(u, wiu[...], preferred_element_type=jnp.float32)
          + bi[...])
    gh = jnp.dot(nf, wht[...], preferred_element_type=jnp.float32) + bh[...]
    r = jax.nn.sigmoid(gi[:, :_D] + gh[:, :_D])
    z = jax.nn.sigmoid(gi[:, _D:2 * _D] + gh[:, _D:2 * _D])
    n = jnp.tanh(gi[:, 2 * _D:] + r * gh[:, 2 * _D:])
    o_ref[...] = (1.0 - z) * n + z * nf


_final_call = pl.pallas_call(
    _final_body,
    grid=(5,),
    in_specs=[
        pl.BlockSpec((2000, _D), lambda i: (i, 0)),
        pl.BlockSpec((2000, _D), lambda i: (i, 0)),
        pl.BlockSpec((2000, _NW), lambda i: (i, 0)),
        pl.BlockSpec((2000, _D), lambda i: (i, 0)),
        pl.BlockSpec((2000, _D), lambda i: (i, 0)),
        pl.BlockSpec((_D, _D), lambda i: (0, 0)),
        pl.BlockSpec((1, _D), lambda i: (0, 0)),
        pl.BlockSpec((_D, 3 * _D), lambda i: (0, 0)),
        pl.BlockSpec((_D, 3 * _D), lambda i: (0, 0)),
        pl.BlockSpec((1, 3 * _D), lambda i: (0, 0)),
        pl.BlockSpec((_D, 3 * _D), lambda i: (0, 0)),
        pl.BlockSpec((1, 3 * _D), lambda i: (0, 0)),
    ],
    out_specs=pl.BlockSpec((2000, _D), lambda i: (i, 0)),
    out_shape=jax.ShapeDtypeStruct((_N, _D), jnp.float32),
)


def kernel(node_features, edge_features, edge_index, W1, b1, W2, b2,
           W_ih, b_ih, W_hh, b_hh):
    nf = node_features
    src = edge_index[0]
    dst = edge_index[1]
    pad_e = _EPAD - _E
    srcp = jnp.concatenate([src, jnp.full((pad_e,), _N, jnp.int32)])
    dstp = jnp.concatenate([dst, jnp.full((pad_e,), _N, jnp.int32)])
    efp = jnp.concatenate(
        [edge_features, jnp.zeros((pad_e, _DE), jnp.float32)])

    w1at = W1[:, :_D].T
    w1bt = W1[:, _D:2 * _D].T
    w1ct = W1[:, 2 * _D:].T

    a_tab, b_tab = _ab_call(nf, w1at, w1bt)
    zpad = jnp.zeros((_NTAB - _N, _D), jnp.bfloat16)
    a_tab = jnp.concatenate([a_tab, zpad])
    b_tab = jnp.concatenate([b_tab, zpad])
    c_rows = _c_call(efp, w1ct, b1.reshape(1, _D))

    sh, deg_all = _get_sc_scatter()(
        a_tab, b_tab, c_rows,
        srcp.reshape(_NW, _NBLK, _BLK), dstp.reshape(_NW, _NBLK, _BLK))

    xy = nf.reshape(2, _N // 2, _D)
    u = _attn_call(xy, xy).reshape(_N, _D)

    wiht = W_ih.T
    h_new = _final_call(
        sh[0, :_N], sh[1, :_N], deg_all[:, :_N].T, u, nf,
        W2.T, b2.reshape(1, _D), wiht[:_D], wiht[_D:],
        b_ih.reshape(1, 3 * _D), W_hh.T, b_hh.reshape(1, 3 * _D))
    return h_new
